# Initial kernel scaffold; baseline (speedup 1.0000x reference)
import jax
import jax.numpy as jnp
from jax.experimental import pallas as pl


def kernel(inputs, s, modulation_s, embeddings, W_sm, b_sm, W1, A1, A1b, W2, A2, A2b, W3, A3, A3b):
    raise NotImplementedError("write your pallas kernel here")


# speedup vs baseline: 2.1106x; 2.1106x over previous
import functools

import jax
import jax.numpy as jnp
import numpy as np
from jax import lax
from jax.experimental import pallas as pl
from jax.experimental.pallas import tpu as pltpu
from jax.experimental.pallas import tpu_sc as plsc

_INPUT_DIM = 4
_NUM_LEVELS = 16
_LEVEL_DIM = 2
_BASE_RES = 16
_LOG2_HASH = 19
_DESIRED_RES = 128
_STYLE_DIM = 256
_HIDDEN = _NUM_LEVELS * _LEVEL_DIM
_OUT_DIM = 64
_BATCH = 4
_PIX = _DESIRED_RES * _DESIRED_RES
_N = _BATCH * _PIX
_SCALE_STEP = float(np.exp2(np.log2(_DESIRED_RES / _BASE_RES) / (_NUM_LEVELS - 1)))
_MAX_PARAMS = 2 ** _LOG2_HASH
_PRIMES = (1, 2654435761, 805459861, 3674653429)

def _level_tables():
    offsets = []
    offset = 0
    for i in range(_NUM_LEVELS):
        res = int(np.ceil(_BASE_RES * _SCALE_STEP ** i))
        params = min(_MAX_PARAMS, (res + 1) ** _INPUT_DIM)
        params = int(np.ceil(params / 8) * 8)
        offsets.append(offset)
        offset += params
    offsets.append(offset)
    levels = []
    S = np.log2(_SCALE_STEP)
    for l in range(_NUM_LEVELS):
        scale = float(np.exp2(l * S) * _BASE_RES - 1.0)
        res = int(np.ceil(scale)) + 1
        hsize = offsets[l + 1] - offsets[l]
        stride_full = (res + 1) ** _INPUT_DIM
        levels.append(dict(scale=scale, R=res + 1, hashed=stride_full > hsize,
                           hsize=hsize, off=offsets[l]))
    return offsets, levels

_OFFSETS, _LEVELS = _level_tables()
_TOTAL_ROWS = _OFFSETS[-1]

_CH = 128
_NCHUNK = _N // _CH
_NC, _NS = 2, 16
_NW = _NC * _NS
_CPW = _NCHUNK // _NW
_ROWS = _NUM_LEVELS * 16
_ENC_W = _CH * _HIDDEN

_I32P = [np.int32(np.uint32(p)) for p in _PRIMES]
_MASK19 = np.int32(_MAX_PARAMS - 1)


def _stage_a_body(inp_ref, sc_ref, idx_ref, w_ref):
    i = pl.program_id(0)
    b = i // (_PIX // (8 * _CH))
    x0 = inp_ref[0]
    x1 = inp_ref[1]
    scb = sc_ref[pl.ds(b, 1), :]
    x2 = scb[0, 0]
    x3 = scb[0, 1]
    for l, lv in enumerate(_LEVELS):
        scale = np.float32(lv["scale"])
        p0 = x0 * scale + 0.5
        p1 = x1 * scale + 0.5
        p2 = x2 * scale + 0.5
        p3 = x3 * scale + 0.5
        g0 = jnp.floor(p0); f0 = p0 - g0
        g1 = jnp.floor(p1); f1 = p1 - g1
        g2 = jnp.floor(p2); f2 = p2 - g2
        g3 = jnp.floor(p3); f3 = p3 - g3
        u0 = g0.astype(jnp.int32)
        u1 = g1.astype(jnp.int32)
        u2 = g2.astype(jnp.int32)
        u3 = g3.astype(jnp.int32)
        w01 = [(1 - f0) * (1 - f1), (1 - f0) * f1, f0 * (1 - f1), f0 * f1]
        w23 = [(1 - f2) * (1 - f3), (1 - f2) * f3, f2 * (1 - f3), f2 * f3]
        off = np.int32(lv["off"])
        if lv["hashed"]:
            a0, b0 = u0, u0 + 1
            a1, b1 = u1 * _I32P[1], (u1 + 1) * _I32P[1]
            a2, b2 = u2 * _I32P[2], (u2 + 1) * _I32P[2]
            a3, b3 = u3 * _I32P[3], (u3 + 1) * _I32P[3]
            t01 = [a0 ^ a1, a0 ^ b1, b0 ^ a1, b0 ^ b1]
            t23 = [a2 ^ a3, a2 ^ b3, b2 ^ a3, b2 ^ b3]
            for r in range(16):
                row = ((t01[r >> 2] ^ t23[r & 3]) & _MASK19) + off
                idx_ref[:, l * 16 + r, :] = row
                w_ref[:, l * 16 + r, :] = w01[r >> 2] * w23[r & 3]
        else:
            R = lv["R"]
            base = u0 + u1 * np.int32(R) + (u2 * np.int32(R * R)
                                            + u3 * np.int32(R * R * R)) + off
            for r in range(16):
                cst = np.int32((r >> 3 & 1) + (r >> 2 & 1) * R
                               + (r >> 1 & 1) * R * R + (r & 1) * R * R * R)
                idx_ref[:, l * 16 + r, :] = base + cst
                w_ref[:, l * 16 + r, :] = w01[r >> 2] * w23[r & 3]


def _run_stage_a(inp_t, s_coords):
    return pl.pallas_call(
        _stage_a_body,
        grid=(_NCHUNK // 8,),
        in_specs=[
            pl.BlockSpec((2, 8, _CH), lambda i: (0, i, 0)),
            pl.BlockSpec((_BATCH, 2), lambda i: (0, 0)),
        ],
        out_specs=[
            pl.BlockSpec((8, _ROWS, _CH), lambda i: (i, 0, 0)),
            pl.BlockSpec((8, _ROWS, _CH), lambda i: (i, 0, 0)),
        ],
        out_shape=[
            jax.ShapeDtypeStruct((_NCHUNK, _ROWS, _CH), jnp.int32),
            jax.ShapeDtypeStruct((_NCHUNK, _ROWS, _CH), jnp.float32),
        ],
    )(inp_t, s_coords)


_HROWS = 128 * _CH


def _stage_b_body(table_hbm, idx_hbm, w_hbm, enc_hbm, idx_v, w_v, rows0_v,
                  rows1_v, enc_v, sem0, sem1):
    wid = lax.axis_index("s") * _NC + lax.axis_index("c")
    iota = lax.iota(jnp.int32, 16)

    def chunk_body(ci, _):
        c = wid * _CPW + ci
        pltpu.sync_copy(idx_hbm.at[c], idx_v)
        pltpu.sync_copy(w_hbm.at[c], w_v)

        def level_body(lv, _):
            cps = []
            for r in range(16):
                rowh = lv * 16 + r
                cps.append(pltpu.async_copy(
                    table_hbm.at[0].at[idx_v.at[rowh]], rows0_v.at[r], sem0))
                cps.append(pltpu.async_copy(
                    table_hbm.at[1].at[idx_v.at[rowh]], rows1_v.at[r], sem1))
            for cp in cps:
                cp.wait()
            col0 = lv * 2
            for q in range(_CH // 16):
                acc0 = jnp.zeros((16,), jnp.float32)
                acc1 = jnp.zeros((16,), jnp.float32)
                for r in range(16):
                    fv0 = rows0_v[r, pl.ds(q * 16, 16)]
                    fv1 = rows1_v[r, pl.ds(q * 16, 16)]
                    wv = w_v[lv * 16 + r, pl.ds(q * 16, 16)]
                    acc0 = acc0 + wv * fv0
                    acc1 = acc1 + wv * fv1
                eidx = (iota + np.int32(q * 16)) * np.int32(_HIDDEN) + col0
                plsc.store_scatter(enc_v, [eidx], acc0)
                plsc.store_scatter(enc_v, [eidx + 1], acc1)
            return 0

        lax.fori_loop(0, _NUM_LEVELS, level_body, 0)
        pltpu.sync_copy(enc_v, enc_hbm.at[pl.ds(c * _ENC_W, _ENC_W)])
        return 0

    lax.fori_loop(0, _CPW, chunk_body, 0)


def _run_stage_b(emb_t, idx3, w):
    mesh = plsc.VectorSubcoreMesh(core_axis_name="c", subcore_axis_name="s")
    kfn = functools.partial(
        pl.kernel,
        mesh=mesh,
        compiler_params=pltpu.CompilerParams(
            use_tc_tiling_on_sc=False, needs_layout_passes=False),
        out_type=jax.ShapeDtypeStruct((_N * _HIDDEN,), jnp.float32),
        scratch_types=[
            pltpu.VMEM((_ROWS, _CH), jnp.int32),
            pltpu.VMEM((_ROWS, _CH), jnp.float32),
            pltpu.VMEM((16, _CH), jnp.float32),
            pltpu.VMEM((16, _CH), jnp.float32),
            pltpu.VMEM((_ENC_W,), jnp.float32),
            pltpu.SemaphoreType.DMA,
            pltpu.SemaphoreType.DMA,
        ],
    )(_stage_b_body)
    return kfn(emb_t, idx3, w)


def _mod_weights(mod_row, W, A, Ab):
    style = jnp.sum(A * mod_row, axis=1)[None, :] * np.float32(
        1.0 / np.sqrt(_STYLE_DIM)) + Ab
    wmod = W * np.float32(1.0 / np.sqrt(W.shape[1])) * style
    demod = lax.rsqrt(jnp.sum(wmod * wmod, axis=1, keepdims=True) + 1e-8)
    return wmod * demod


def _stage_c_body(enc_ref, mod_ref, w1_ref, a1_ref, a1b_ref, w2_ref, a2_ref,
                  a2b_ref, w3_ref, a3_ref, a3b_ref, out_ref):
    mod_row = mod_ref[0]
    h = enc_ref[0]
    for (w_ref, a_ref, ab_ref, act) in (
            (w1_ref, a1_ref, a1b_ref, True),
            (w2_ref, a2_ref, a2b_ref, True),
            (w3_ref, a3_ref, a3b_ref, False)):
        wd = _mod_weights(mod_row, w_ref[...], a_ref[...], ab_ref[...])
        h = lax.dot_general(h, wd, (((1,), (1,)), ((), ())),
                            preferred_element_type=jnp.float32,
                            precision=lax.Precision.HIGHEST)
        if act:
            h = jnp.where(h > 0, h, h * np.float32(0.01))
    out_ref[0] = h


def _run_stage_c(enc3, mod_s, W1, A1, A1b2, W2, A2, A2b2, W3, A3, A3b2):
    full = lambda shape: pl.BlockSpec(shape, lambda b, j: tuple(0 for _ in shape))
    pblk = 4096
    return pl.pallas_call(
        _stage_c_body,
        grid=(_BATCH, _PIX // pblk),
        in_specs=[
            pl.BlockSpec((1, pblk, _HIDDEN), lambda b, j: (b, j, 0)),
            pl.BlockSpec((1, 1, _STYLE_DIM), lambda b, j: (b, 0, 0)),
            full((_HIDDEN, _HIDDEN)), full((_HIDDEN, _STYLE_DIM)),
            full((1, _HIDDEN)),
            full((_HIDDEN, _HIDDEN)), full((_HIDDEN, _STYLE_DIM)),
            full((1, _HIDDEN)),
            full((_OUT_DIM, _HIDDEN)), full((_HIDDEN, _STYLE_DIM)),
            full((1, _HIDDEN)),
        ],
        out_specs=pl.BlockSpec((1, pblk, _OUT_DIM), lambda b, j: (b, j, 0)),
        out_shape=jax.ShapeDtypeStruct((_BATCH, _PIX, _OUT_DIM), jnp.float32),
    )(enc3, mod_s.reshape(_BATCH, 1, _STYLE_DIM), W1, A1, A1b2, W2, A2, A2b2,
      W3, A3, A3b2)


def kernel(inputs, s, modulation_s, embeddings, W_sm, b_sm, W1, A1, A1b,
           W2, A2, A2b, W3, A3, A3b):
    inp_t = inputs.T.reshape(2, _NCHUNK, _CH)
    s_coords = jax.nn.sigmoid(s @ W_sm.T * (1.0 / np.sqrt(_STYLE_DIM)) + b_sm)
    idx, w = _run_stage_a(inp_t, s_coords)
    enc_flat = _run_stage_b(embeddings.T, idx, w)
    enc3 = enc_flat.reshape(_BATCH, _PIX, _HIDDEN)
    out = _run_stage_c(enc3, modulation_s, W1, A1, A1b.reshape(1, -1),
                       W2, A2, A2b.reshape(1, -1), W3, A3, A3b.reshape(1, -1))
    return out.reshape(_N, _OUT_DIM)
